# Initial kernel scaffold; baseline (speedup 1.0000x reference)
import functools

import jax
import jax.numpy as jnp
from jax.experimental import pallas as pl
from jax.experimental.pallas import tpu as pltpu
from jax.experimental.pallas import tpu_sc as plsc

_T = 1.0
_EPS = 1e-8
_LANES = 16
_CHUNK = 8192


def _table_body(theta_ref, tlo_ref, thi_ref, y0_ref, slope_ref):
    th = theta_ref[...]
    m = jnp.max(th)
    e = jnp.exp(th - m)
    w = e / jnp.sum(e)
    inc = w * _T
    cs = jnp.cumsum(inc, axis=1)
    y0 = cs - inc
    denom = thi_ref[...] - tlo_ref[...] + _EPS
    y0_ref[...] = y0
    slope_ref[...] = (cs - y0) / denom


def _make_tables(theta, times):
    km1 = theta.shape[0]
    out_shape = (jax.ShapeDtypeStruct((1, km1), jnp.float32),
                 jax.ShapeDtypeStruct((1, km1), jnp.float32))
    y0, slope = pl.pallas_call(_table_body, out_shape=out_shape)(
        theta.reshape(1, km1),
        times[:-1].reshape(1, km1),
        times[1:].reshape(1, km1))
    return y0.reshape(km1), slope.reshape(km1)


def _make_interp(nb, ch, km1):
    mesh = plsc.VectorSubcoreMesh(core_axis_name="c", subcore_axis_name="s")
    scale = float(km1) / _T
    inv_scale = _T / float(km1)

    @functools.partial(
        pl.kernel, mesh=mesh,
        out_type=(jax.ShapeDtypeStruct((nb, ch), jnp.float32),
                  jax.ShapeDtypeStruct((nb, ch), jnp.float32)),
        scratch_types=[pltpu.VMEM((km1,), jnp.float32),
                       pltpu.VMEM((km1,), jnp.float32)],
    )
    def k(t_hbm, y0_hbm, slope_hbm, tau_hbm, dtau_hbm, y0_v, slope_v):
        pltpu.sync_copy(y0_hbm, y0_v)
        pltpu.sync_copy(slope_hbm, slope_v)

        def body(t_vmem, tau_vmem, dtau_vmem):
            @pl.loop(0, ch, step=_LANES)
            def _(i):
                tv = t_vmem[0, pl.ds(i, _LANES)]
                x = tv * scale
                xi = x.astype(jnp.int32)
                xf = xi.astype(jnp.float32)
                ceil = jnp.where(x > xf, xi + 1, xi)
                j = jnp.minimum(jnp.maximum(ceil - 1, 0), km1 - 1)
                y0 = plsc.load_gather(y0_v, [j])
                s = plsc.load_gather(slope_v, [j])
                t0 = j.astype(jnp.float32) * inv_scale
                tau_vmem[0, pl.ds(i, _LANES)] = y0 + (tv - t0) * s
                dtau_vmem[0, pl.ds(i, _LANES)] = s

        pltpu.emit_pipeline(
            body,
            grid=(nb,),
            in_specs=[pl.BlockSpec((1, ch), lambda i: (i, 0))],
            out_specs=[pl.BlockSpec((1, ch), lambda i: (i, 0)),
                       pl.BlockSpec((1, ch), lambda i: (i, 0))],
            core_axis_name=("c", "s"),
            dimension_semantics=(pltpu.PARALLEL,),
        )(t_hbm, tau_hbm, dtau_hbm)

    return k


def kernel(t, theta, times):
    n = t.shape[0]
    km1 = theta.shape[0]
    y0, slope = _make_tables(theta, times)
    ch = _CHUNK
    nb = n // ch
    t2 = t.reshape(nb, ch)
    tau, dtau = _make_interp(nb, ch, km1)(t2, y0, slope)
    return tau.reshape(t.shape), dtau.reshape(t.shape)


# speedup vs baseline: 2057.2783x; 2057.2783x over previous
import dataclasses
import functools

import jax
import jax.numpy as jnp
from jax.experimental import pallas as pl
from jax.experimental.pallas import tpu as pltpu
from jax.experimental.pallas import tpu_sc as plsc

_T = 1.0
_EPS = 1e-8
_LANES = 16
_CHUNK = 8192


def _table_body(theta_ref, tlo_ref, thi_ref, y0_ref, slope_ref):
    th = theta_ref[...]
    m = jnp.max(th)
    e = jnp.exp(th - m)
    w = e / jnp.sum(e)
    inc = w * _T
    km1 = inc.shape[1]
    row = jax.lax.broadcasted_iota(jnp.int32, (km1, km1), 0)
    col = jax.lax.broadcasted_iota(jnp.int32, (km1, km1), 1)
    tri = jnp.where(row <= col, 1.0, 0.0).astype(jnp.float32)
    cs = jax.lax.dot_general(inc, tri, (((1,), (0,)), ((), ())),
                             precision=jax.lax.Precision.HIGHEST,
                             preferred_element_type=jnp.float32)
    y0 = cs - inc
    denom = thi_ref[...] - tlo_ref[...] + _EPS
    y0_ref[...] = y0
    slope_ref[...] = (cs - y0) / denom


def _make_tables(theta, times):
    km1 = theta.shape[0]
    out_shape = (jax.ShapeDtypeStruct((1, km1), jnp.float32),
                 jax.ShapeDtypeStruct((1, km1), jnp.float32))
    y0, slope = pl.pallas_call(_table_body, out_shape=out_shape)(
        theta.reshape(1, km1),
        times[:-1].reshape(1, km1),
        times[1:].reshape(1, km1))
    return y0.reshape(km1), slope.reshape(km1)


def _make_interp(nb, ch, km1):
    mesh = plsc.VectorSubcoreMesh(core_axis_name="c", subcore_axis_name="s")
    scale = float(km1) / _T
    inv_scale = _T / float(km1)

    cp = pltpu.CompilerParams()
    if "needs_layout_passes" in pltpu.CompilerParams.__dataclass_fields__:
        cp = dataclasses.replace(cp, needs_layout_passes=False)

    @functools.partial(
        pl.kernel, mesh=mesh,
        out_type=(jax.ShapeDtypeStruct((nb, ch), jnp.float32),
                  jax.ShapeDtypeStruct((nb, ch), jnp.float32)),
        scratch_types=[pltpu.VMEM((km1,), jnp.float32),
                       pltpu.VMEM((km1,), jnp.float32)],
        compiler_params=cp,
    )
    def k(t_hbm, y0_hbm, slope_hbm, tau_hbm, dtau_hbm, y0_v, slope_v):
        pltpu.sync_copy(y0_hbm, y0_v)
        pltpu.sync_copy(slope_hbm, slope_v)

        def body(t_vmem, tau_vmem, dtau_vmem):
            @pl.loop(0, ch, step=_LANES)
            def _(i):
                tv = t_vmem[0, pl.ds(i, _LANES)]
                x = tv * scale
                xi = x.astype(jnp.int32)
                xf = xi.astype(jnp.float32)
                ceil = jnp.where(x > xf, xi + 1, xi)
                j = jnp.minimum(jnp.maximum(ceil - 1, 0), km1 - 1)
                y0 = plsc.load_gather(y0_v, [j])
                s = plsc.load_gather(slope_v, [j])
                t0 = j.astype(jnp.float32) * inv_scale
                tau_vmem[0, pl.ds(i, _LANES)] = y0 + (tv - t0) * s
                dtau_vmem[0, pl.ds(i, _LANES)] = s

        pltpu.emit_pipeline(
            body,
            grid=(nb,),
            in_specs=[pl.BlockSpec((1, ch), lambda i: (i, 0))],
            out_specs=[pl.BlockSpec((1, ch), lambda i: (i, 0)),
                       pl.BlockSpec((1, ch), lambda i: (i, 0))],
            core_axis_name=("c", "s"),
            dimension_semantics=(pltpu.PARALLEL,),
        )(t_hbm, tau_hbm, dtau_hbm)

    return k


def kernel(t, theta, times):
    n = t.shape[0]
    km1 = theta.shape[0]
    y0, slope = _make_tables(theta, times)
    ch = _CHUNK
    nb = n // ch
    t2 = t.reshape(nb, ch)
    tau, dtau = _make_interp(nb, ch, km1)(t2, y0, slope)
    return tau.reshape(t.shape), dtau.reshape(t.shape)


# speedup vs baseline: 4752.4301x; 2.3101x over previous
import dataclasses
import functools

import jax
import jax.numpy as jnp
from jax.experimental import pallas as pl
from jax.experimental.pallas import tpu as pltpu
from jax.experimental.pallas import tpu_sc as plsc

_T = 1.0
_EPS = 1e-8
_LANES = 16
_CHUNK = 8192


def _table_body(theta_ref, tlo_ref, thi_ref, y0_ref, slope_ref):
    th = theta_ref[...]
    m = jnp.max(th)
    e = jnp.exp(th - m)
    w = e / jnp.sum(e)
    inc = w * _T
    km1 = inc.shape[1]
    row = jax.lax.broadcasted_iota(jnp.int32, (km1, km1), 0)
    col = jax.lax.broadcasted_iota(jnp.int32, (km1, km1), 1)
    tri = jnp.where(row <= col, 1.0, 0.0).astype(jnp.float32)
    cs = jax.lax.dot_general(inc, tri, (((1,), (0,)), ((), ())),
                             precision=jax.lax.Precision.HIGHEST,
                             preferred_element_type=jnp.float32)
    y0 = cs - inc
    denom = thi_ref[...] - tlo_ref[...] + _EPS
    y0_ref[...] = y0
    slope_ref[...] = (cs - y0) / denom


def _make_tables(theta, times):
    km1 = theta.shape[0]
    out_shape = (jax.ShapeDtypeStruct((1, km1), jnp.float32),
                 jax.ShapeDtypeStruct((1, km1), jnp.float32))
    y0, slope = pl.pallas_call(_table_body, out_shape=out_shape)(
        theta.reshape(1, km1),
        times[:-1].reshape(1, km1),
        times[1:].reshape(1, km1))
    return y0.reshape(km1), slope.reshape(km1)


def _make_interp(nb, ch, km1):
    mesh = plsc.VectorSubcoreMesh(core_axis_name="c", subcore_axis_name="s")
    scale = float(km1) / _T
    inv_scale = _T / float(km1)

    cp = pltpu.CompilerParams()
    if "needs_layout_passes" in pltpu.CompilerParams.__dataclass_fields__:
        cp = dataclasses.replace(cp, needs_layout_passes=False)

    @functools.partial(
        pl.kernel, mesh=mesh,
        out_type=(jax.ShapeDtypeStruct((nb, ch), jnp.float32),
                  jax.ShapeDtypeStruct((nb, ch), jnp.float32)),
        scratch_types=[pltpu.VMEM((km1,), jnp.float32),
                       pltpu.VMEM((km1,), jnp.float32)],
        compiler_params=cp,
    )
    def k(t_hbm, y0_hbm, slope_hbm, tau_hbm, dtau_hbm, y0_v, slope_v):
        pltpu.sync_copy(y0_hbm, y0_v)
        pltpu.sync_copy(slope_hbm, slope_v)

        def body(t_vmem, tau_vmem, dtau_vmem):
            @plsc.parallel_loop(0, ch, step=_LANES, unroll=8)
            def _(i):
                tv = t_vmem[0, pl.ds(i, _LANES)]
                x = tv * scale
                xi = x.astype(jnp.int32)
                xf = xi.astype(jnp.float32)
                ceil = jnp.where(x > xf, xi + 1, xi)
                j = jnp.minimum(jnp.maximum(ceil - 1, 0), km1 - 1)
                y0 = plsc.load_gather(y0_v, [j])
                s = plsc.load_gather(slope_v, [j])
                t0 = j.astype(jnp.float32) * inv_scale
                tau_vmem[0, pl.ds(i, _LANES)] = y0 + (tv - t0) * s
                dtau_vmem[0, pl.ds(i, _LANES)] = s

        pltpu.emit_pipeline(
            body,
            grid=(nb,),
            in_specs=[pl.BlockSpec((1, ch), lambda i: (i, 0))],
            out_specs=[pl.BlockSpec((1, ch), lambda i: (i, 0)),
                       pl.BlockSpec((1, ch), lambda i: (i, 0))],
            core_axis_name=("c", "s"),
            dimension_semantics=(pltpu.PARALLEL,),
        )(t_hbm, tau_hbm, dtau_hbm)

    return k


def kernel(t, theta, times):
    n = t.shape[0]
    km1 = theta.shape[0]
    y0, slope = _make_tables(theta, times)
    ch = _CHUNK
    nb = n // ch
    t2 = t.reshape(nb, ch)
    tau, dtau = _make_interp(nb, ch, km1)(t2, y0, slope)
    return tau.reshape(t.shape), dtau.reshape(t.shape)


# speedup vs baseline: 11189.6470x; 2.3545x over previous
import dataclasses
import functools

import jax
import jax.numpy as jnp
from jax.experimental import pallas as pl
from jax.experimental.pallas import tpu as pltpu
from jax.experimental.pallas import tpu_sc as plsc

_T = 1.0
_EPS = 1e-8
_LANES = 16
_CHUNK = 16384


def _table_body(theta_ref, tlo_ref, thi_ref, b_ref, slope_ref):
    th = theta_ref[...]
    m = jnp.max(th)
    e = jnp.exp(th - m)
    w = e / jnp.sum(e)
    inc = w * _T
    km1 = inc.shape[1]
    row = jax.lax.broadcasted_iota(jnp.int32, (km1, km1), 0)
    col = jax.lax.broadcasted_iota(jnp.int32, (km1, km1), 1)
    tri = jnp.where(row <= col, 1.0, 0.0).astype(jnp.float32)
    cs = jax.lax.dot_general(inc, tri, (((1,), (0,)), ((), ())),
                             precision=jax.lax.Precision.HIGHEST,
                             preferred_element_type=jnp.float32)
    y0 = cs - inc
    tlo = tlo_ref[...]
    denom = thi_ref[...] - tlo + _EPS
    slope = (cs - y0) / denom
    slope_ref[...] = slope
    b_ref[...] = y0 - tlo * slope


def _make_tables(theta, times):
    km1 = theta.shape[0]
    out_shape = (jax.ShapeDtypeStruct((1, km1), jnp.float32),
                 jax.ShapeDtypeStruct((1, km1), jnp.float32))
    b, slope = pl.pallas_call(_table_body, out_shape=out_shape)(
        theta.reshape(1, km1),
        times[:-1].reshape(1, km1),
        times[1:].reshape(1, km1))
    return b.reshape(km1), slope.reshape(km1)


def _make_interp(n, ch, km1):
    mesh = plsc.VectorSubcoreMesh(core_axis_name="c", subcore_axis_name="s")
    scale = float(km1) / _T

    cp = pltpu.CompilerParams()
    if "needs_layout_passes" in pltpu.CompilerParams.__dataclass_fields__:
        cp = dataclasses.replace(cp, needs_layout_passes=False)

    @functools.partial(
        pl.kernel, mesh=mesh,
        out_type=(jax.ShapeDtypeStruct((n,), jnp.float32),
                  jax.ShapeDtypeStruct((n,), jnp.float32)),
        scratch_types=[pltpu.VMEM((km1,), jnp.float32),
                       pltpu.VMEM((km1,), jnp.float32)],
        compiler_params=cp,
    )
    def k(t_hbm, b_hbm, slope_hbm, tau_hbm, dtau_hbm, b_v, slope_v):
        pltpu.sync_copy(b_hbm, b_v)
        pltpu.sync_copy(slope_hbm, slope_v)

        def body(t_vmem, tau_vmem, dtau_vmem):
            @plsc.parallel_loop(0, ch, step=_LANES, unroll=8)
            def _(i):
                tv = t_vmem[pl.ds(i, _LANES)]
                x = tv * scale
                xi = x.astype(jnp.int32)
                xf = xi.astype(jnp.float32)
                j = jnp.maximum(jnp.where(x == xf, xi - 1, xi), 0)
                b = plsc.load_gather(b_v, [j])
                s = plsc.load_gather(slope_v, [j])
                tau_vmem[pl.ds(i, _LANES)] = b + tv * s
                dtau_vmem[pl.ds(i, _LANES)] = s

        pltpu.emit_pipeline(
            body,
            grid=(n // ch,),
            in_specs=[pl.BlockSpec((ch,), lambda i: (i,))],
            out_specs=[pl.BlockSpec((ch,), lambda i: (i,)),
                       pl.BlockSpec((ch,), lambda i: (i,))],
            core_axis_name=("c", "s"),
            dimension_semantics=(pltpu.PARALLEL,),
        )(t_hbm, tau_hbm, dtau_hbm)

    return k


def kernel(t, theta, times):
    n = t.shape[0]
    km1 = theta.shape[0]
    b, slope = _make_tables(theta, times)
    tau, dtau = _make_interp(n, _CHUNK, km1)(t, b, slope)
    return tau, dtau
